# baseline (speedup 1.0000x reference)
import functools

import jax
import jax.numpy as jnp
from jax.experimental import pallas as pl
from jax.experimental.pallas import tpu as pltpu

_INTERPRET = False

_LOG2E = 1.4426950408889634
_CONTRACT_LAST = (((1,), (1,)), ((), ()))
_VMEM_LIM = 100 * 1024 * 1024


def _bf(x):
    return x.astype(jnp.bfloat16)


def _gather_body(ids_ref, emb_ref, out_ref, sem):
    n = out_ref.shape[0]
    b = pl.program_id(0)

    def issue(i, _):
        idx = ids_ref[b * n + i]
        pltpu.make_async_copy(emb_ref.at[pl.ds(idx, 1), :],
                              out_ref.at[pl.ds(i, 1), :], sem).start()
        return 0

    jax.lax.fori_loop(0, n, issue, 0)

    def drain(i, _):
        pltpu.make_async_copy(emb_ref.at[pl.ds(0, 1), :],
                              out_ref.at[pl.ds(0, 1), :], sem).wait()
        return 0

    jax.lax.fori_loop(0, n, drain, 0)


def _embed_gather(ids_flat, embed, *, nsplit):
    m = ids_flat.shape[0]
    dm = embed.shape[1]
    return pl.pallas_call(
        _gather_body,
        grid=(nsplit,),
        out_shape=jax.ShapeDtypeStruct((m, dm), jnp.float32),
        in_specs=[pl.BlockSpec(memory_space=pltpu.SMEM),
                  pl.BlockSpec(memory_space=pl.ANY)],
        out_specs=pl.BlockSpec((m // nsplit, dm), lambda b: (b, 0)),
        scratch_shapes=[pltpu.SemaphoreType.DMA],
        compiler_params=pltpu.CompilerParams(
            dimension_semantics=("parallel",),
        ),
        name="embed_gather",
        interpret=_INTERPRET,
    )(ids_flat, embed)


def _silu(v):
    return v * jax.nn.sigmoid(v)


def _inproj_body(x_ref, nw_ref, wu_ref, wz_ref, cw_ref, cb_ref,
                 u_ref, zs_ref):
    xv = x_ref[...]
    ms = jnp.mean(xv * xv, axis=-1, keepdims=True)
    hb = _bf(xv * jax.lax.rsqrt(ms + 1e-5) * nw_ref[...])

    xz_u = jax.lax.dot_general(hb, _bf(wu_ref[...]), _CONTRACT_LAST,
                               preferred_element_type=jnp.float32)
    xz_z = jax.lax.dot_general(hb, _bf(wz_ref[...]), _CONTRACT_LAST,
                               preferred_element_type=jnp.float32)

    m, w = xz_u.shape
    dc = cw_ref.shape[0]
    uc = xz_u * cw_ref[dc - 1:dc, :]
    for s in range(1, dc):
        shifted = jnp.concatenate(
            [jnp.zeros((s, w), jnp.float32), xz_u[:-s, :]], axis=0)
        uc = uc + shifted * cw_ref[dc - 1 - s:dc - s, :]
    u_ref[...] = _silu(uc + cb_ref[...])
    zs_ref[...] = _silu(xz_z)


def _inproj(x, norm_w_l, in_proj_w_l, cw_l, cb_l, *, di, seg, wchunk):
    m, dm = x.shape
    nb = m // seg
    nc = di // wchunk
    dc = cw_l.shape[0]
    return pl.pallas_call(
        _inproj_body,
        grid=(nb, nc),
        in_specs=[
            pl.BlockSpec((seg, dm), lambda b, c: (b, 0)),
            pl.BlockSpec((1, dm), lambda b, c: (0, 0)),
            pl.BlockSpec((wchunk, dm), lambda b, c: (c, 0)),
            pl.BlockSpec((wchunk, dm), lambda b, c, _nc=nc: (c + _nc, 0)),
            pl.BlockSpec((dc, wchunk), lambda b, c: (0, c)),
            pl.BlockSpec((1, wchunk), lambda b, c: (0, c)),
        ],
        out_specs=[
            pl.BlockSpec((seg, wchunk), lambda b, c: (b, c)),
            pl.BlockSpec((seg, wchunk), lambda b, c: (b, c)),
        ],
        out_shape=[jax.ShapeDtypeStruct((m, di), jnp.float32),
                   jax.ShapeDtypeStruct((m, di), jnp.float32)],
        compiler_params=pltpu.CompilerParams(
            dimension_semantics=("parallel", "arbitrary"),
            vmem_limit_bytes=_VMEM_LIM,
        ),
        name="rms_inproj_conv",
        interpret=_INTERPRET,
    )(x, norm_w_l, in_proj_w_l, in_proj_w_l, cw_l, cb_l)


def _xdt_body(u_ref, wdtr_ref, wb_ref, wc_ref, wdt_ref, dtb_ref,
              dt_ref, dtu_ref, bm_ref, cm_ref):
    uv = u_ref[...]
    ub = _bf(uv)
    dtr = jax.lax.dot_general(ub, _bf(wdtr_ref[...]), _CONTRACT_LAST,
                              preferred_element_type=jnp.float32)
    bm_ref[...] = jax.lax.dot_general(ub, _bf(wb_ref[...]), _CONTRACT_LAST,
                                      preferred_element_type=jnp.float32)
    cm_ref[...] = jax.lax.dot_general(ub, _bf(wc_ref[...]), _CONTRACT_LAST,
                                      preferred_element_type=jnp.float32)
    dtx = jax.lax.dot_general(_bf(dtr), _bf(wdt_ref[...]), _CONTRACT_LAST,
                              preferred_element_type=jnp.float32)
    dtx = dtx + dtb_ref[...]
    dt = jnp.where(dtx > 20.0, dtx, jnp.log1p(jnp.exp(dtx)))
    dt_ref[...] = dt
    dtu_ref[...] = dt * uv


def _xdt(u, wdtr, wb, wc, wdt, dtb, *, ds, seg):
    m, di = u.shape
    nb = m // seg
    dtrk = wdtr.shape[0]
    return pl.pallas_call(
        _xdt_body,
        grid=(nb,),
        in_specs=[
            pl.BlockSpec((seg, di), lambda b: (b, 0)),
            pl.BlockSpec((dtrk, di), lambda b: (0, 0)),
            pl.BlockSpec((ds, di), lambda b: (0, 0)),
            pl.BlockSpec((ds, di), lambda b: (0, 0)),
            pl.BlockSpec((di, dtrk), lambda b: (0, 0)),
            pl.BlockSpec((1, di), lambda b: (0, 0)),
        ],
        out_specs=[
            pl.BlockSpec((seg, di), lambda b: (b, 0)),
            pl.BlockSpec((seg, di), lambda b: (b, 0)),
            pl.BlockSpec((seg, ds), lambda b: (b, 0)),
            pl.BlockSpec((seg, ds), lambda b: (b, 0)),
        ],
        out_shape=[jax.ShapeDtypeStruct((m, di), jnp.float32),
                   jax.ShapeDtypeStruct((m, di), jnp.float32),
                   jax.ShapeDtypeStruct((m, ds), jnp.float32),
                   jax.ShapeDtypeStruct((m, ds), jnp.float32)],
        compiler_params=pltpu.CompilerParams(
            dimension_semantics=("parallel",),
            vmem_limit_bytes=_VMEM_LIM,
        ),
        name="xproj_dtproj",
        interpret=_INTERPRET,
    )(u, wdtr, wb, wc, wdt, dtb)


def _scan_body(dt_ref, dtu_ref, u_ref, zs_ref, alog_ref, d_ref,
               bm_ref, cm_ref, yg_ref):
    ds = alog_ref.shape[0]
    w = alog_ref.shape[1]
    nblk = dt_ref.shape[0]
    a_sc = (-_LOG2E) * jnp.exp(alog_ref[...])
    dvec = d_ref[...]

    def body(blk, h):
        dt8 = dt_ref[blk]
        dtu8 = dtu_ref[blk]
        bc8 = bm_ref[blk]
        cc8 = cm_ref[blk]
        ys = []
        for j in range(8):
            dt_row = dt8[j:j + 1, :]
            a = jnp.exp2(a_sc * dt_row)
            dbu = bc8[:, j:j + 1] * dtu8[j:j + 1, :]
            h = a * h + dbu
            ys.append(jnp.sum(cc8[:, j:j + 1] * h, axis=0, keepdims=True))
        y8 = jnp.concatenate(ys, axis=0)
        yg8 = (y8 + u_ref[blk] * dvec) * zs_ref[blk]
        yg_ref[blk] = yg8
        return h

    jax.lax.fori_loop(0, nblk, body, jnp.zeros((ds, w), jnp.float32))


def _scan(dt3, dtu3, u3, zs3, alogT, dvec2, bm_t8, cm_t8, *, seg, wchunk):
    mb, eight, di = dt3.shape
    m = mb * eight
    nb = m // seg
    sb = seg // 8
    ds = alogT.shape[0]
    nc = di // wchunk
    yg3 = pl.pallas_call(
        _scan_body,
        grid=(nb, nc),
        in_specs=[
            pl.BlockSpec((sb, 8, wchunk), lambda b, c: (b, 0, c)),
            pl.BlockSpec((sb, 8, wchunk), lambda b, c: (b, 0, c)),
            pl.BlockSpec((sb, 8, wchunk), lambda b, c: (b, 0, c)),
            pl.BlockSpec((sb, 8, wchunk), lambda b, c: (b, 0, c)),
            pl.BlockSpec((ds, wchunk), lambda b, c: (0, c)),
            pl.BlockSpec((1, wchunk), lambda b, c: (0, c)),
            pl.BlockSpec((sb, ds, 8), lambda b, c: (b, 0, 0)),
            pl.BlockSpec((sb, ds, 8), lambda b, c: (b, 0, 0)),
        ],
        out_specs=pl.BlockSpec((sb, 8, wchunk), lambda b, c: (b, 0, c)),
        out_shape=jax.ShapeDtypeStruct((mb, 8, di), jnp.float32),
        compiler_params=pltpu.CompilerParams(
            dimension_semantics=("parallel", "arbitrary"),
            vmem_limit_bytes=_VMEM_LIM,
        ),
        name="selective_scan",
        interpret=_INTERPRET,
    )(dt3, dtu3, u3, zs3, alogT, dvec2, bm_t8, cm_t8)
    return yg3


def _outproj_body(yg_ref, w_ref, x_ref, o_ref):
    o_ref[...] = x_ref[...] + jax.lax.dot_general(
        _bf(yg_ref[...]), _bf(w_ref[...]), _CONTRACT_LAST,
        preferred_element_type=jnp.float32)


def _outproj(yg, w_out, x, *, seg):
    m, dm = x.shape
    di = yg.shape[1]
    nb = m // seg
    return pl.pallas_call(
        _outproj_body,
        grid=(nb,),
        in_specs=[
            pl.BlockSpec((seg, di), lambda b: (b, 0)),
            pl.BlockSpec((dm, di), lambda b: (0, 0)),
            pl.BlockSpec((seg, dm), lambda b: (b, 0)),
        ],
        out_specs=pl.BlockSpec((seg, dm), lambda b: (b, 0)),
        out_shape=jax.ShapeDtypeStruct((m, dm), jnp.float32),
        compiler_params=pltpu.CompilerParams(
            dimension_semantics=("parallel",),
            vmem_limit_bytes=_VMEM_LIM,
        ),
        name="outproj_residual",
        interpret=_INTERPRET,
    )(yg, w_out, x)


def _rmsf_body(x_ref, w_ref, o_ref):
    xv = x_ref[...]
    ms = jnp.mean(xv * xv, axis=-1, keepdims=True)
    o_ref[...] = _bf(xv * jax.lax.rsqrt(ms + 1e-5) * w_ref[...])


def _rms_final(x, w):
    m, dm = x.shape
    return pl.pallas_call(
        _rmsf_body,
        grid=(2,),
        in_specs=[
            pl.BlockSpec((m // 2, dm), lambda b: (b, 0)),
            pl.BlockSpec((1, dm), lambda b: (0, 0)),
        ],
        out_specs=pl.BlockSpec((m // 2, dm), lambda b: (b, 0)),
        out_shape=jax.ShapeDtypeStruct((m, dm), jnp.bfloat16),
        compiler_params=pltpu.CompilerParams(
            dimension_semantics=("parallel",),
        ),
        name="rms_final",
        interpret=_INTERPRET,
    )(x, w.reshape(1, dm))


def _lmhead_body(h_ref, e_ref, o_ref):
    o_ref[...] = jax.lax.dot_general(
        h_ref[...], _bf(e_ref[...]), _CONTRACT_LAST,
        preferred_element_type=jnp.float32)


def _lmhead(hf, embed, *, vtile):
    m, dm = hf.shape
    v = embed.shape[0]
    nv = v // vtile
    nvh = nv // 2
    return pl.pallas_call(
        _lmhead_body,
        grid=(2, nvh),
        in_specs=[
            pl.BlockSpec((m, dm), lambda b, i: (0, 0)),
            pl.BlockSpec((vtile, dm), lambda b, i, _h=nvh: (b * _h + i, 0)),
        ],
        out_specs=pl.BlockSpec((m, vtile), lambda b, i, _h=nvh: (0, b * _h + i)),
        out_shape=jax.ShapeDtypeStruct((m, v), jnp.float32),
        compiler_params=pltpu.CompilerParams(
            dimension_semantics=("parallel", "arbitrary"),
            vmem_limit_bytes=_VMEM_LIM,
        ),
        name="lm_head",
        interpret=_INTERPRET,
    )(hf, embed)


def kernel(input_ids, embed, norm_w, in_proj_w, conv_w, conv_b, x_proj_w,
           dt_proj_w, dt_proj_b, A_log, D, out_proj_w, norm_f_w):
    bsz, seg = input_ids.shape
    v, dm = embed.shape
    nl, di, ds = A_log.shape
    dtr = dt_proj_w.shape[2]
    m = bsz * seg
    wchunk = 512

    cw = jnp.swapaxes(conv_w[:, :, 0, :], 1, 2)
    alogT = jnp.swapaxes(A_log, 1, 2)
    wdtr = x_proj_w[:, :dtr, :]
    wb = x_proj_w[:, dtr:dtr + ds, :]
    wc = x_proj_w[:, dtr + ds:, :]

    x = _embed_gather(input_ids.reshape(m), embed, nsplit=bsz)

    for l in range(nl):
        u, zs = _inproj(x, norm_w[l].reshape(1, dm), in_proj_w[l],
                        cw[l], conv_b[l].reshape(1, di),
                        di=di, seg=seg, wchunk=wchunk)
        dt, dtu, bm, cm = _xdt(u, wdtr[l], wb[l], wc[l], dt_proj_w[l],
                               dt_proj_b[l].reshape(1, di), ds=ds, seg=seg)
        dt3 = dt.reshape(m // 8, 8, di)
        dtu3 = dtu.reshape(m // 8, 8, di)
        u3 = u.reshape(m // 8, 8, di)
        zs3 = zs.reshape(m // 8, 8, di)
        bm_t8 = jnp.swapaxes(bm.reshape(m // 8, 8, ds), 1, 2)
        cm_t8 = jnp.swapaxes(cm.reshape(m // 8, 8, ds), 1, 2)
        yg3 = _scan(dt3, dtu3, u3, zs3, alogT[l], D[l].reshape(1, di),
                    bm_t8, cm_t8, seg=seg, wchunk=wchunk)
        x = _outproj(yg3.reshape(m, di), out_proj_w[l], x, seg=seg)

    hf = _rms_final(x, norm_f_w)
    logits = _lmhead(hf, embed, vtile=640)
    return logits.reshape(bsz, seg, v)


# speedup vs baseline: 1.1130x; 1.1130x over previous
import functools

import jax
import jax.numpy as jnp
from jax.experimental import pallas as pl
from jax.experimental.pallas import tpu as pltpu

_INTERPRET = False

_LOG2E = 1.4426950408889634
_CONTRACT_LAST = (((1,), (1,)), ((), ()))
_VMEM_LIM = 100 * 1024 * 1024


def _bf(x):
    return x.astype(jnp.bfloat16)


def _silu(v):
    return v * jax.nn.sigmoid(v)


def _gather_body(ids_ref, emb_ref, out_ref, sem):
    n = out_ref.shape[0]

    def issue(i, _):
        idx = ids_ref[i]
        pltpu.make_async_copy(emb_ref.at[pl.ds(idx, 1), :],
                              out_ref.at[pl.ds(i, 1), :], sem).start()
        return 0

    jax.lax.fori_loop(0, n, issue, 0)

    def drain(i, _):
        pltpu.make_async_copy(emb_ref.at[pl.ds(0, 1), :],
                              out_ref.at[pl.ds(0, 1), :], sem).wait()
        return 0

    jax.lax.fori_loop(0, n, drain, 0)


def _embed_gather(ids_flat, embed):
    m = ids_flat.shape[0]
    dm = embed.shape[1]
    return pl.pallas_call(
        _gather_body,
        out_shape=jax.ShapeDtypeStruct((m, dm), jnp.float32),
        in_specs=[pl.BlockSpec(memory_space=pltpu.SMEM),
                  pl.BlockSpec(memory_space=pl.ANY)],
        out_specs=pl.BlockSpec(memory_space=pltpu.VMEM),
        scratch_shapes=[pltpu.SemaphoreType.DMA],
        name="embed_gather",
        interpret=_INTERPRET,
    )(ids_flat, embed)


def _proj_chunk(hb, wu_ref, wz_ref, cw_ref, cb_ref, pos, seg):
    xz_u = jax.lax.dot_general(hb, _bf(wu_ref[...]), _CONTRACT_LAST,
                               preferred_element_type=jnp.float32)
    xz_z = jax.lax.dot_general(hb, _bf(wz_ref[...]), _CONTRACT_LAST,
                               preferred_element_type=jnp.float32)
    m, w = xz_u.shape
    dc = cw_ref.shape[0]
    uc = xz_u * cw_ref[dc - 1:dc, :]
    for s in range(1, dc):
        shifted = jnp.concatenate(
            [jnp.zeros((s, w), jnp.float32), xz_u[:-s, :]], axis=0)
        shifted = jnp.where(pos >= s, shifted, 0.0)
        uc = uc + shifted * cw_ref[dc - 1 - s:dc - s, :]
    return _silu(uc + cb_ref[...]), _silu(xz_z)


def _rms_bf16(xv, nw):
    ms = jnp.mean(xv * xv, axis=-1, keepdims=True)
    return _bf(xv * jax.lax.rsqrt(ms + 1e-5) * nw)


def _inproj_body(x_ref, nw_ref, wu_ref, wz_ref, cw_ref, cb_ref,
                 u_ref, zs_ref, hb_ref, *, seg):
    @pl.when(pl.program_id(0) == 0)
    def _():
        hb_ref[...] = _rms_bf16(x_ref[...], nw_ref[...])

    m = x_ref.shape[0]
    row = jax.lax.broadcasted_iota(jnp.int32, (m, 1), 0)
    pos = jax.lax.rem(row, seg)
    u, zs = _proj_chunk(hb_ref[...], wu_ref, wz_ref, cw_ref, cb_ref, pos, seg)
    u_ref[...] = u
    zs_ref[...] = zs


def _inproj(x, norm_w_l, in_proj_w_l, cw_l, cb_l, *, di, seg, wchunk):
    m, dm = x.shape
    nc = di // wchunk
    dc = cw_l.shape[0]
    kern = functools.partial(_inproj_body, seg=seg)
    return pl.pallas_call(
        kern,
        grid=(nc,),
        in_specs=[
            pl.BlockSpec((m, dm), lambda c: (0, 0)),
            pl.BlockSpec((1, dm), lambda c: (0, 0)),
            pl.BlockSpec((wchunk, dm), lambda c: (c, 0)),
            pl.BlockSpec((wchunk, dm), lambda c, _nc=nc: (c + _nc, 0)),
            pl.BlockSpec((dc, wchunk), lambda c: (0, c)),
            pl.BlockSpec((1, wchunk), lambda c: (0, c)),
        ],
        out_specs=[
            pl.BlockSpec((m, wchunk), lambda c: (0, c)),
            pl.BlockSpec((m, wchunk), lambda c: (0, c)),
        ],
        out_shape=[jax.ShapeDtypeStruct((m, di), jnp.float32),
                   jax.ShapeDtypeStruct((m, di), jnp.float32)],
        scratch_shapes=[pltpu.VMEM((m, dm), jnp.bfloat16)],
        compiler_params=pltpu.CompilerParams(
            dimension_semantics=("arbitrary",),
            vmem_limit_bytes=_VMEM_LIM,
        ),
        name="rms_inproj_conv",
        interpret=_INTERPRET,
    )(x, norm_w_l, in_proj_w_l, in_proj_w_l, cw_l, cb_l)


def _outin_body(yg_ref, wo_ref, x_ref, nw_ref, wu_ref, wz_ref, cw_ref, cb_ref,
                xo_ref, u_ref, zs_ref, hb_ref, *, seg):
    @pl.when(pl.program_id(0) == 0)
    def _():
        xn = x_ref[...] + jax.lax.dot_general(
            _bf(yg_ref[...]), _bf(wo_ref[...]), _CONTRACT_LAST,
            preferred_element_type=jnp.float32)
        xo_ref[...] = xn
        hb_ref[...] = _rms_bf16(xn, nw_ref[...])

    m = x_ref.shape[0]
    row = jax.lax.broadcasted_iota(jnp.int32, (m, 1), 0)
    pos = jax.lax.rem(row, seg)
    u, zs = _proj_chunk(hb_ref[...], wu_ref, wz_ref, cw_ref, cb_ref, pos, seg)
    u_ref[...] = u
    zs_ref[...] = zs


def _outin(yg, w_out, x, norm_w_n, in_proj_w_n, cw_n, cb_n, *, di, seg,
           wchunk):
    m, dm = x.shape
    nc = di // wchunk
    dc = cw_n.shape[0]
    kern = functools.partial(_outin_body, seg=seg)
    return pl.pallas_call(
        kern,
        grid=(nc,),
        in_specs=[
            pl.BlockSpec((m, di), lambda c: (0, 0)),
            pl.BlockSpec((dm, di), lambda c: (0, 0)),
            pl.BlockSpec((m, dm), lambda c: (0, 0)),
            pl.BlockSpec((1, dm), lambda c: (0, 0)),
            pl.BlockSpec((wchunk, dm), lambda c: (c, 0)),
            pl.BlockSpec((wchunk, dm), lambda c, _nc=nc: (c + _nc, 0)),
            pl.BlockSpec((dc, wchunk), lambda c: (0, c)),
            pl.BlockSpec((1, wchunk), lambda c: (0, c)),
        ],
        out_specs=[
            pl.BlockSpec((m, dm), lambda c: (0, 0)),
            pl.BlockSpec((m, wchunk), lambda c: (0, c)),
            pl.BlockSpec((m, wchunk), lambda c: (0, c)),
        ],
        out_shape=[jax.ShapeDtypeStruct((m, dm), jnp.float32),
                   jax.ShapeDtypeStruct((m, di), jnp.float32),
                   jax.ShapeDtypeStruct((m, di), jnp.float32)],
        scratch_shapes=[pltpu.VMEM((m, dm), jnp.bfloat16)],
        compiler_params=pltpu.CompilerParams(
            dimension_semantics=("arbitrary",),
            vmem_limit_bytes=_VMEM_LIM,
        ),
        name="outproj_inproj",
        interpret=_INTERPRET,
    )(yg, w_out, x, norm_w_n, in_proj_w_n, in_proj_w_n, cw_n, cb_n)


def _outrms_body(yg_ref, wo_ref, x_ref, wf_ref, hf_ref):
    xn = x_ref[...] + jax.lax.dot_general(
        _bf(yg_ref[...]), _bf(wo_ref[...]), _CONTRACT_LAST,
        preferred_element_type=jnp.float32)
    hf_ref[...] = _rms_bf16(xn, wf_ref[...])


def _outrms(yg, w_out, x, norm_f_w):
    m, dm = x.shape
    di = yg.shape[1]
    return pl.pallas_call(
        _outrms_body,
        out_shape=jax.ShapeDtypeStruct((m, dm), jnp.bfloat16),
        compiler_params=pltpu.CompilerParams(
            vmem_limit_bytes=_VMEM_LIM,
        ),
        name="outproj_rms_final",
        interpret=_INTERPRET,
    )(yg, w_out, x, norm_f_w.reshape(1, dm))


def _xdt_body(u_ref, wdtr_ref, wb_ref, wc_ref, wdt_ref, dtb_ref,
              dt_ref, bm_ref, cm_ref):
    ub = _bf(u_ref[...])
    dtr = jax.lax.dot_general(ub, _bf(wdtr_ref[...]), _CONTRACT_LAST,
                              preferred_element_type=jnp.float32)
    bm_ref[...] = jax.lax.dot_general(ub, _bf(wb_ref[...]), _CONTRACT_LAST,
                                      preferred_element_type=jnp.float32)
    cm_ref[...] = jax.lax.dot_general(ub, _bf(wc_ref[...]), _CONTRACT_LAST,
                                      preferred_element_type=jnp.float32)
    dtx = jax.lax.dot_general(_bf(dtr), _bf(wdt_ref[...]), _CONTRACT_LAST,
                              preferred_element_type=jnp.float32)
    dtx = dtx + dtb_ref[...]
    dt_ref[...] = jnp.where(dtx > 20.0, dtx, jnp.log1p(jnp.exp(dtx)))


def _xdt(u, wdtr, wb, wc, wdt, dtb, *, ds):
    m, di = u.shape
    return pl.pallas_call(
        _xdt_body,
        out_shape=[jax.ShapeDtypeStruct((m, di), jnp.float32),
                   jax.ShapeDtypeStruct((m, ds), jnp.float32),
                   jax.ShapeDtypeStruct((m, ds), jnp.float32)],
        compiler_params=pltpu.CompilerParams(
            vmem_limit_bytes=_VMEM_LIM,
        ),
        name="xproj_dtproj",
        interpret=_INTERPRET,
    )(u, wdtr, wb, wc, wdt, dtb)


def _scan_body(dt_ref, u_ref, zs_ref, alog_ref, d_ref,
               bm_ref, cm_ref, yg_ref, *, seg):
    ds = alog_ref.shape[0]
    w = alog_ref.shape[1]
    nblk = seg // 8
    nbatch = dt_ref.shape[0] // nblk
    a_sc = (-_LOG2E) * jnp.exp(alog_ref[...])
    dvec = d_ref[...]

    def batch_block(base, h):
        dt8 = dt_ref[base]
        u8 = u_ref[base]
        bc8 = bm_ref[base]
        cc8 = cm_ref[base]
        ys = []
        for j in range(8):
            dt_row = dt8[j:j + 1, :]
            a = jnp.exp2(a_sc * dt_row)
            dbu = bc8[:, j:j + 1] * (dt_row * u8[j:j + 1, :])
            h = a * h + dbu
            ys.append(jnp.sum(cc8[:, j:j + 1] * h, axis=0, keepdims=True))
        y8 = jnp.concatenate(ys, axis=0)
        yg_ref[base] = (y8 + u8 * dvec) * zs_ref[base]
        return h

    def body(blk, carry):
        return tuple(
            batch_block(b * nblk + blk, carry[b]) for b in range(nbatch))

    z = jnp.zeros((ds, w), jnp.float32)
    jax.lax.fori_loop(0, nblk, body, (z,) * nbatch)


def _scan(dt3, u3, zs3, alogT, dvec2, bm_t8, cm_t8, *, seg, wchunk):
    mb, eight, di = dt3.shape
    ds = alogT.shape[0]
    nc = di // wchunk
    kern = functools.partial(_scan_body, seg=seg)
    yg3 = pl.pallas_call(
        kern,
        grid=(nc,),
        in_specs=[
            pl.BlockSpec((mb, 8, wchunk), lambda c: (0, 0, c)),
            pl.BlockSpec((mb, 8, wchunk), lambda c: (0, 0, c)),
            pl.BlockSpec((mb, 8, wchunk), lambda c: (0, 0, c)),
            pl.BlockSpec((ds, wchunk), lambda c: (0, c)),
            pl.BlockSpec((1, wchunk), lambda c: (0, c)),
            pl.BlockSpec((mb, ds, 8), lambda c: (0, 0, 0)),
            pl.BlockSpec((mb, ds, 8), lambda c: (0, 0, 0)),
        ],
        out_specs=pl.BlockSpec((mb, 8, wchunk), lambda c: (0, 0, c)),
        out_shape=jax.ShapeDtypeStruct((mb, 8, di), jnp.float32),
        compiler_params=pltpu.CompilerParams(
            dimension_semantics=("arbitrary",),
            vmem_limit_bytes=_VMEM_LIM,
        ),
        name="selective_scan",
        interpret=_INTERPRET,
    )(dt3, u3, zs3, alogT, dvec2, bm_t8, cm_t8)
    return yg3


def _lmhead_body(h_ref, e_ref, o_ref):
    o_ref[...] = jax.lax.dot_general(
        h_ref[...], _bf(e_ref[...]), _CONTRACT_LAST,
        preferred_element_type=jnp.float32)


def _lmhead(hf, embed, *, vtile):
    m, dm = hf.shape
    v = embed.shape[0]
    nv = v // vtile
    return pl.pallas_call(
        _lmhead_body,
        grid=(nv,),
        in_specs=[
            pl.BlockSpec((m, dm), lambda i: (0, 0)),
            pl.BlockSpec((vtile, dm), lambda i: (i, 0)),
        ],
        out_specs=pl.BlockSpec((m, vtile), lambda i: (0, i)),
        out_shape=jax.ShapeDtypeStruct((m, v), jnp.float32),
        compiler_params=pltpu.CompilerParams(
            dimension_semantics=("arbitrary",),
            vmem_limit_bytes=_VMEM_LIM,
        ),
        name="lm_head",
        interpret=_INTERPRET,
    )(hf, embed)


def kernel(input_ids, embed, norm_w, in_proj_w, conv_w, conv_b, x_proj_w,
           dt_proj_w, dt_proj_b, A_log, D, out_proj_w, norm_f_w):
    bsz, seg = input_ids.shape
    v, dm = embed.shape
    nl, di, ds = A_log.shape
    dtr = dt_proj_w.shape[2]
    m = bsz * seg
    wchunk = 512

    cw = jnp.swapaxes(conv_w[:, :, 0, :], 1, 2)
    alogT = jnp.swapaxes(A_log, 1, 2)
    wdtr = x_proj_w[:, :dtr, :]
    wb = x_proj_w[:, dtr:dtr + ds, :]
    wc = x_proj_w[:, dtr + ds:, :]

    x = _embed_gather(input_ids.reshape(m), embed)
    u, zs = _inproj(x, norm_w[0].reshape(1, dm), in_proj_w[0],
                    cw[0], conv_b[0].reshape(1, di),
                    di=di, seg=seg, wchunk=wchunk)

    for l in range(nl):
        dt, bm, cm = _xdt(u, wdtr[l], wb[l], wc[l], dt_proj_w[l],
                          dt_proj_b[l].reshape(1, di), ds=ds)
        dt3 = dt.reshape(m // 8, 8, di)
        u3 = u.reshape(m // 8, 8, di)
        zs3 = zs.reshape(m // 8, 8, di)
        bm_t8 = jnp.swapaxes(bm.reshape(m // 8, 8, ds), 1, 2)
        cm_t8 = jnp.swapaxes(cm.reshape(m // 8, 8, ds), 1, 2)
        yg3 = _scan(dt3, u3, zs3, alogT[l], D[l].reshape(1, di),
                    bm_t8, cm_t8, seg=seg, wchunk=wchunk)
        yg = yg3.reshape(m, di)
        if l + 1 < nl:
            x, u, zs = _outin(yg, out_proj_w[l], x, norm_w[l + 1].reshape(1, dm),
                              in_proj_w[l + 1], cw[l + 1],
                              conv_b[l + 1].reshape(1, di),
                              di=di, seg=seg, wchunk=wchunk)
        else:
            hf = _outrms(yg, out_proj_w[l], x, norm_f_w)

    logits = _lmhead(hf, embed, vtile=1280)
    return logits.reshape(bsz, seg, v)


# speedup vs baseline: 1.2644x; 1.1360x over previous
import functools

import jax
import jax.numpy as jnp
from jax.experimental import pallas as pl
from jax.experimental.pallas import tpu as pltpu

_INTERPRET = False

_LOG2E = 1.4426950408889634
_CONTRACT_LAST = (((1,), (1,)), ((), ()))
_VMEM_LIM = 100 * 1024 * 1024


def _bf(x):
    return x.astype(jnp.bfloat16)


def _silu(v):
    return v * jax.nn.sigmoid(v)


def _gather_body(ids_ref, emb_ref, out_ref, sem):
    n = out_ref.shape[0]

    def issue(i, _):
        idx = ids_ref[i]
        pltpu.make_async_copy(emb_ref.at[pl.ds(idx, 1), :],
                              out_ref.at[pl.ds(i, 1), :], sem).start()
        return 0

    jax.lax.fori_loop(0, n, issue, 0)

    def drain(i, _):
        pltpu.make_async_copy(emb_ref.at[pl.ds(0, 1), :],
                              out_ref.at[pl.ds(0, 1), :], sem).wait()
        return 0

    jax.lax.fori_loop(0, n, drain, 0)


def _embed_gather(ids_flat, embed):
    m = ids_flat.shape[0]
    dm = embed.shape[1]
    return pl.pallas_call(
        _gather_body,
        out_shape=jax.ShapeDtypeStruct((m, dm), jnp.float32),
        in_specs=[pl.BlockSpec(memory_space=pltpu.SMEM),
                  pl.BlockSpec(memory_space=pl.ANY)],
        out_specs=pl.BlockSpec(memory_space=pltpu.VMEM),
        scratch_shapes=[pltpu.SemaphoreType.DMA],
        name="embed_gather",
        interpret=_INTERPRET,
    )(ids_flat, embed)


def _proj_chunk(hb, wu_ref, wz_ref, cw_ref, cb_ref, pos, seg):
    xz_u = jax.lax.dot_general(hb, _bf(wu_ref[...]), _CONTRACT_LAST,
                               preferred_element_type=jnp.float32)
    xz_z = jax.lax.dot_general(hb, _bf(wz_ref[...]), _CONTRACT_LAST,
                               preferred_element_type=jnp.float32)
    m, w = xz_u.shape
    dc = cw_ref.shape[0]
    uc = xz_u * cw_ref[dc - 1:dc, :]
    for s in range(1, dc):
        shifted = jnp.concatenate(
            [jnp.zeros((s, w), jnp.float32), xz_u[:-s, :]], axis=0)
        shifted = jnp.where(pos >= s, shifted, 0.0)
        uc = uc + shifted * cw_ref[dc - 1 - s:dc - s, :]
    return _silu(uc + cb_ref[...]), _silu(xz_z)


def _rms_bf16(xv, nw):
    ms = jnp.mean(xv * xv, axis=-1, keepdims=True)
    return _bf(xv * jax.lax.rsqrt(ms + 1e-5) * nw)


def _inproj_body(x_ref, nw_ref, wu_ref, wz_ref, cw_ref, cb_ref,
                 u_ref, zs_ref, hb_ref, *, seg):
    @pl.when(pl.program_id(0) == 0)
    def _():
        hb_ref[...] = _rms_bf16(x_ref[...], nw_ref[...])

    m = x_ref.shape[0]
    row = jax.lax.broadcasted_iota(jnp.int32, (m, 1), 0)
    pos = jax.lax.rem(row, seg)
    u, zs = _proj_chunk(hb_ref[...], wu_ref, wz_ref, cw_ref, cb_ref, pos, seg)
    u_ref[...] = u
    zs_ref[...] = zs


def _inproj(x, norm_w_l, in_proj_w_l, cw_l, cb_l, *, di, seg, wchunk):
    m, dm = x.shape
    nc = di // wchunk
    dc = cw_l.shape[0]
    kern = functools.partial(_inproj_body, seg=seg)
    return pl.pallas_call(
        kern,
        grid=(nc,),
        in_specs=[
            pl.BlockSpec((m, dm), lambda c: (0, 0)),
            pl.BlockSpec((1, dm), lambda c: (0, 0)),
            pl.BlockSpec((wchunk, dm), lambda c: (c, 0)),
            pl.BlockSpec((wchunk, dm), lambda c, _nc=nc: (c + _nc, 0)),
            pl.BlockSpec((dc, wchunk), lambda c: (0, c)),
            pl.BlockSpec((1, wchunk), lambda c: (0, c)),
        ],
        out_specs=[
            pl.BlockSpec((m, wchunk), lambda c: (0, c)),
            pl.BlockSpec((m, wchunk), lambda c: (0, c)),
        ],
        out_shape=[jax.ShapeDtypeStruct((m, di), jnp.float32),
                   jax.ShapeDtypeStruct((m, di), jnp.float32)],
        scratch_shapes=[pltpu.VMEM((m, dm), jnp.bfloat16)],
        compiler_params=pltpu.CompilerParams(
            dimension_semantics=("arbitrary",),
            vmem_limit_bytes=_VMEM_LIM,
        ),
        name="rms_inproj_conv",
        interpret=_INTERPRET,
    )(x, norm_w_l, in_proj_w_l, in_proj_w_l, cw_l, cb_l)


def _outin_body(yg_ref, wo_ref, x_ref, nw_ref, wu_ref, wz_ref, cw_ref, cb_ref,
                xo_ref, u_ref, zs_ref, hb_ref, *, seg):
    @pl.when(pl.program_id(0) == 0)
    def _():
        xn = x_ref[...] + jax.lax.dot_general(
            _bf(yg_ref[...]), _bf(wo_ref[...]), _CONTRACT_LAST,
            preferred_element_type=jnp.float32)
        xo_ref[...] = xn
        hb_ref[...] = _rms_bf16(xn, nw_ref[...])

    m = x_ref.shape[0]
    row = jax.lax.broadcasted_iota(jnp.int32, (m, 1), 0)
    pos = jax.lax.rem(row, seg)
    u, zs = _proj_chunk(hb_ref[...], wu_ref, wz_ref, cw_ref, cb_ref, pos, seg)
    u_ref[...] = u
    zs_ref[...] = zs


def _outin(yg, w_out, x, norm_w_n, in_proj_w_n, cw_n, cb_n, *, di, seg,
           wchunk):
    m, dm = x.shape
    nc = di // wchunk
    dc = cw_n.shape[0]
    kern = functools.partial(_outin_body, seg=seg)
    return pl.pallas_call(
        kern,
        grid=(nc,),
        in_specs=[
            pl.BlockSpec((m, di), lambda c: (0, 0)),
            pl.BlockSpec((dm, di), lambda c: (0, 0)),
            pl.BlockSpec((m, dm), lambda c: (0, 0)),
            pl.BlockSpec((1, dm), lambda c: (0, 0)),
            pl.BlockSpec((wchunk, dm), lambda c: (c, 0)),
            pl.BlockSpec((wchunk, dm), lambda c, _nc=nc: (c + _nc, 0)),
            pl.BlockSpec((dc, wchunk), lambda c: (0, c)),
            pl.BlockSpec((1, wchunk), lambda c: (0, c)),
        ],
        out_specs=[
            pl.BlockSpec((m, dm), lambda c: (0, 0)),
            pl.BlockSpec((m, wchunk), lambda c: (0, c)),
            pl.BlockSpec((m, wchunk), lambda c: (0, c)),
        ],
        out_shape=[jax.ShapeDtypeStruct((m, dm), jnp.float32),
                   jax.ShapeDtypeStruct((m, di), jnp.float32),
                   jax.ShapeDtypeStruct((m, di), jnp.float32)],
        scratch_shapes=[pltpu.VMEM((m, dm), jnp.bfloat16)],
        compiler_params=pltpu.CompilerParams(
            dimension_semantics=("arbitrary",),
            vmem_limit_bytes=_VMEM_LIM,
        ),
        name="outproj_inproj",
        interpret=_INTERPRET,
    )(yg, w_out, x, norm_w_n, in_proj_w_n, in_proj_w_n, cw_n, cb_n)


def _outrms_body(yg_ref, wo_ref, x_ref, wf_ref, hf_ref):
    xn = x_ref[...] + jax.lax.dot_general(
        _bf(yg_ref[...]), _bf(wo_ref[...]), _CONTRACT_LAST,
        preferred_element_type=jnp.float32)
    hf_ref[...] = _rms_bf16(xn, wf_ref[...])


def _outrms(yg, w_out, x, norm_f_w):
    m, dm = x.shape
    di = yg.shape[1]
    return pl.pallas_call(
        _outrms_body,
        out_shape=jax.ShapeDtypeStruct((m, dm), jnp.bfloat16),
        compiler_params=pltpu.CompilerParams(
            vmem_limit_bytes=_VMEM_LIM,
        ),
        name="outproj_rms_final",
        interpret=_INTERPRET,
    )(yg, w_out, x, norm_f_w.reshape(1, dm))


def _xdtscan_body(u_ref, zs_ref, wdtr_ref, wb_ref, wc_ref, wdt_ref, dtb_ref,
                  alog_ref, d_ref, yg_ref,
                  dts_ref, bmt_ref, cmt_ref, bm8_ref, cm8_ref, *, seg):
    mb = u_ref.shape[0]
    di = u_ref.shape[2]
    m = mb * 8
    ds = alog_ref.shape[0]
    nblk = seg // 8
    nbatch = m // seg

    ub = _bf(u_ref[...].reshape(m, di))
    dtr = jax.lax.dot_general(ub, _bf(wdtr_ref[...]), _CONTRACT_LAST,
                              preferred_element_type=jnp.float32)
    bmt_ref[...] = jax.lax.dot_general(_bf(wb_ref[...]), ub, _CONTRACT_LAST,
                                       preferred_element_type=jnp.float32)
    cmt_ref[...] = jax.lax.dot_general(_bf(wc_ref[...]), ub, _CONTRACT_LAST,
                                       preferred_element_type=jnp.float32)
    dtx = jax.lax.dot_general(_bf(dtr), _bf(wdt_ref[...]), _CONTRACT_LAST,
                              preferred_element_type=jnp.float32)
    dtx = dtx + dtb_ref[...]
    dt = jnp.where(dtx > 20.0, dtx, jnp.log1p(jnp.exp(dtx)))
    dts_ref[...] = dt.reshape(mb, 8, di)
    for i in range(mb):
        bm8_ref[i] = bmt_ref[:, 8 * i:8 * i + 8]
        cm8_ref[i] = cmt_ref[:, 8 * i:8 * i + 8]

    a_sc = (-_LOG2E) * jnp.exp(alog_ref[...])
    dvec = d_ref[...]

    def batch_block(base, h):
        dt8 = dts_ref[base]
        u8 = u_ref[base]
        bc8 = bm8_ref[base]
        cc8 = cm8_ref[base]
        ys = []
        for j in range(8):
            dt_row = dt8[j:j + 1, :]
            a = jnp.exp2(a_sc * dt_row)
            dbu = bc8[:, j:j + 1] * (dt_row * u8[j:j + 1, :])
            h = a * h + dbu
            ys.append(jnp.sum(cc8[:, j:j + 1] * h, axis=0, keepdims=True))
        y8 = jnp.concatenate(ys, axis=0)
        yg_ref[base] = (y8 + u8 * dvec) * zs_ref[base]
        return h

    def body(blk, carry):
        return tuple(
            batch_block(b * nblk + blk, carry[b]) for b in range(nbatch))

    z = jnp.zeros((ds, di), jnp.float32)
    jax.lax.fori_loop(0, nblk, body, (z,) * nbatch)


def _xdtscan(u3, zs3, wdtr, wb, wc, wdt, dtb, alogT, dvec2, *, seg):
    mb, eight, di = u3.shape
    m = mb * 8
    ds = alogT.shape[0]
    kern = functools.partial(_xdtscan_body, seg=seg)
    return pl.pallas_call(
        kern,
        out_shape=jax.ShapeDtypeStruct((mb, 8, di), jnp.float32),
        scratch_shapes=[
            pltpu.VMEM((mb, 8, di), jnp.float32),
            pltpu.VMEM((ds, m), jnp.float32),
            pltpu.VMEM((ds, m), jnp.float32),
            pltpu.VMEM((mb, ds, 8), jnp.float32),
            pltpu.VMEM((mb, ds, 8), jnp.float32),
        ],
        compiler_params=pltpu.CompilerParams(
            vmem_limit_bytes=_VMEM_LIM,
        ),
        name="xdt_scan",
        interpret=_INTERPRET,
    )(u3, zs3, wdtr, wb, wc, wdt, dtb, alogT, dvec2)


def _lmhead_body(h_ref, e_ref, o_ref):
    o_ref[...] = jax.lax.dot_general(
        h_ref[...], _bf(e_ref[...]), _CONTRACT_LAST,
        preferred_element_type=jnp.float32)


def _lmhead(hf, embed, *, vtile):
    m, dm = hf.shape
    v = embed.shape[0]
    nv = v // vtile
    return pl.pallas_call(
        _lmhead_body,
        grid=(nv,),
        in_specs=[
            pl.BlockSpec((m, dm), lambda i: (0, 0)),
            pl.BlockSpec((vtile, dm), lambda i: (i, 0)),
        ],
        out_specs=pl.BlockSpec((m, vtile), lambda i: (0, i)),
        out_shape=jax.ShapeDtypeStruct((m, v), jnp.float32),
        compiler_params=pltpu.CompilerParams(
            dimension_semantics=("arbitrary",),
            vmem_limit_bytes=_VMEM_LIM,
        ),
        name="lm_head",
        interpret=_INTERPRET,
    )(hf, embed)


def kernel(input_ids, embed, norm_w, in_proj_w, conv_w, conv_b, x_proj_w,
           dt_proj_w, dt_proj_b, A_log, D, out_proj_w, norm_f_w):
    bsz, seg = input_ids.shape
    v, dm = embed.shape
    nl, di, ds = A_log.shape
    dtr = dt_proj_w.shape[2]
    m = bsz * seg
    wchunk = 512

    cw = jnp.swapaxes(conv_w[:, :, 0, :], 1, 2)
    alogT = jnp.swapaxes(A_log, 1, 2)
    wdtr = x_proj_w[:, :dtr, :]
    wb = x_proj_w[:, dtr:dtr + ds, :]
    wc = x_proj_w[:, dtr + ds:, :]

    x = _embed_gather(input_ids.reshape(m), embed)
    u, zs = _inproj(x, norm_w[0].reshape(1, dm), in_proj_w[0],
                    cw[0], conv_b[0].reshape(1, di),
                    di=di, seg=seg, wchunk=wchunk)

    for l in range(nl):
        yg3 = _xdtscan(u.reshape(m // 8, 8, di), zs.reshape(m // 8, 8, di),
                       wdtr[l], wb[l], wc[l], dt_proj_w[l],
                       dt_proj_b[l].reshape(1, di), alogT[l],
                       D[l].reshape(1, di), seg=seg)
        yg = yg3.reshape(m, di)
        if l + 1 < nl:
            x, u, zs = _outin(yg, out_proj_w[l], x, norm_w[l + 1].reshape(1, dm),
                              in_proj_w[l + 1], cw[l + 1],
                              conv_b[l + 1].reshape(1, di),
                              di=di, seg=seg, wchunk=wchunk)
        else:
            hf = _outrms(yg, out_proj_w[l], x, norm_f_w)

    logits = _lmhead(hf, embed, vtile=1280)
    return logits.reshape(bsz, seg, v)


# speedup vs baseline: 1.5803x; 1.2498x over previous
import functools

import jax
import jax.numpy as jnp
from jax.experimental import pallas as pl
from jax.experimental.pallas import tpu as pltpu

_INTERPRET = False

_LOG2E = 1.4426950408889634
_CONTRACT_LAST = (((1,), (1,)), ((), ()))
_VMEM_LIM = 110 * 1024 * 1024


def _bf(x):
    return x.astype(jnp.bfloat16)


def _silu(v):
    return v * jax.nn.sigmoid(v)


def _rms_bf16(xv, nw):
    ms = jnp.mean(xv * xv, axis=-1, keepdims=True)
    return _bf(xv * jax.lax.rsqrt(ms + 1e-5) * nw)


def _gather_body(ids_ref, emb_ref, out_ref, sem):
    n = out_ref.shape[0]

    def issue(i, _):
        idx = ids_ref[i]
        pltpu.make_async_copy(emb_ref.at[pl.ds(idx, 1), :],
                              out_ref.at[pl.ds(i, 1), :], sem).start()
        return 0

    jax.lax.fori_loop(0, n, issue, 0)

    def drain(i, _):
        pltpu.make_async_copy(emb_ref.at[pl.ds(0, 1), :],
                              out_ref.at[pl.ds(0, 1), :], sem).wait()
        return 0

    jax.lax.fori_loop(0, n, drain, 0)


def _embed_gather(ids_flat, embed):
    m = ids_flat.shape[0]
    dm = embed.shape[1]
    return pl.pallas_call(
        _gather_body,
        out_shape=jax.ShapeDtypeStruct((m, dm), jnp.float32),
        in_specs=[pl.BlockSpec(memory_space=pltpu.SMEM),
                  pl.BlockSpec(memory_space=pl.ANY)],
        out_specs=pl.BlockSpec(memory_space=pltpu.VMEM),
        scratch_shapes=[pltpu.SemaphoreType.DMA],
        name="embed_gather",
        interpret=_INTERPRET,
    )(ids_flat, embed)


def _layers_body(x0_ref, nw_ref, win_ref, cw_ref, cb_ref, wdtr_ref, wb_ref,
                 wc_ref, wdt_ref, dtb_ref, alog_ref, d_ref, wo_ref, wf_ref,
                 hf_ref,
                 x_ref, u_ref, zs_ref, dts_ref, yg_ref,
                 bmt_ref, cmt_ref, bm8_ref, cm8_ref,
                 *, seg, nl):
    l = pl.program_id(0)
    mb = u_ref.shape[0]
    di = u_ref.shape[2]
    m = mb * 8
    ds = alog_ref.shape[1]
    nblk = seg // 8
    nbatch = m // seg

    @pl.when(l == 0)
    def _():
        x_ref[...] = x0_ref[...]

    hb = _rms_bf16(x_ref[...], nw_ref[0])
    xz_u = jax.lax.dot_general(hb, win_ref[0, 0:di], _CONTRACT_LAST,
                               preferred_element_type=jnp.float32)
    xz_z = jax.lax.dot_general(hb, win_ref[0, di:2 * di], _CONTRACT_LAST,
                               preferred_element_type=jnp.float32)
    dc = cw_ref.shape[1]
    row = jax.lax.broadcasted_iota(jnp.int32, (m, 1), 0)
    pos = jax.lax.rem(row, seg)
    uc = xz_u * cw_ref[0, dc - 1:dc, :]
    for s in range(1, dc):
        shifted = jnp.concatenate(
            [jnp.zeros((s, di), jnp.float32), xz_u[:-s, :]], axis=0)
        shifted = jnp.where(pos >= s, shifted, 0.0)
        uc = uc + shifted * cw_ref[0, dc - 1 - s:dc - s, :]
    u = _silu(uc + cb_ref[0])
    u_ref[...] = u.reshape(mb, 8, di)
    zs_ref[...] = _bf(_silu(xz_z)).reshape(mb, 8, di)

    ub = _bf(u)
    dtr = jax.lax.dot_general(ub, wdtr_ref[0], _CONTRACT_LAST,
                              preferred_element_type=jnp.float32)
    bmt_ref[...] = jax.lax.dot_general(wb_ref[0], ub, _CONTRACT_LAST,
                                       preferred_element_type=jnp.float32)
    cmt_ref[...] = jax.lax.dot_general(wc_ref[0], ub, _CONTRACT_LAST,
                                       preferred_element_type=jnp.float32)
    dtx = jax.lax.dot_general(_bf(dtr), wdt_ref[0], _CONTRACT_LAST,
                              preferred_element_type=jnp.float32)
    dtx = dtx + dtb_ref[0]
    dt = jnp.where(dtx > 20.0, dtx, jnp.log1p(jnp.exp(dtx)))
    dts_ref[...] = dt.reshape(mb, 8, di)
    for i in range(mb):
        bm8_ref[i] = bmt_ref[:, 8 * i:8 * i + 8]
        cm8_ref[i] = cmt_ref[:, 8 * i:8 * i + 8]

    a_sc = (-_LOG2E) * jnp.exp(alog_ref[0])
    dvec = d_ref[0]

    def batch_block(base, h):
        dt8 = dts_ref[base]
        u8 = u_ref[base]
        bc8 = bm8_ref[base]
        cc8 = cm8_ref[base]
        ys = []
        for j in range(8):
            dt_row = dt8[j:j + 1, :]
            a = jnp.exp2(a_sc * dt_row)
            dbu = bc8[:, j:j + 1] * (dt_row * u8[j:j + 1, :])
            h = a * h + dbu
            ys.append(jnp.sum(cc8[:, j:j + 1] * h, axis=0, keepdims=True))
        y8 = jnp.concatenate(ys, axis=0)
        yg_ref[base] = _bf((y8 + u8 * dvec) *
                           zs_ref[base].astype(jnp.float32))
        return h

    def body(blk, carry):
        return tuple(
            batch_block(b * nblk + blk, carry[b]) for b in range(nbatch))

    z = jnp.zeros((ds, di), jnp.float32)
    jax.lax.fori_loop(0, nblk, body, (z,) * nbatch)

    xn = x_ref[...] + jax.lax.dot_general(
        yg_ref[...].reshape(m, di), wo_ref[0], _CONTRACT_LAST,
        preferred_element_type=jnp.float32)
    x_ref[...] = xn

    @pl.when(l == nl - 1)
    def _():
        hf_ref[...] = _rms_bf16(xn, wf_ref[0])


def _layers(x0, norm_w, win_bf, cw, cb, wdtr_bf, wb_bf, wc_bf, wdt_bf, dtb,
            alogT, dmat, wo_bf, norm_f_w, *, seg):
    m, dm = x0.shape
    nl, ds, di = alogT.shape
    dtrk = wdt_bf.shape[2]
    dc = cw.shape[1]
    mb = m // 8
    kern = functools.partial(_layers_body, seg=seg, nl=nl)
    return pl.pallas_call(
        kern,
        grid=(nl,),
        in_specs=[
            pl.BlockSpec((m, dm), lambda l: (0, 0)),
            pl.BlockSpec((1, 1, dm), lambda l: (l, 0, 0)),
            pl.BlockSpec((1, 2 * di, dm), lambda l: (l, 0, 0)),
            pl.BlockSpec((1, dc, di), lambda l: (l, 0, 0)),
            pl.BlockSpec((1, 1, di), lambda l: (l, 0, 0)),
            pl.BlockSpec((1, dtrk, di), lambda l: (l, 0, 0)),
            pl.BlockSpec((1, ds, di), lambda l: (l, 0, 0)),
            pl.BlockSpec((1, ds, di), lambda l: (l, 0, 0)),
            pl.BlockSpec((1, di, dtrk), lambda l: (l, 0, 0)),
            pl.BlockSpec((1, 1, di), lambda l: (l, 0, 0)),
            pl.BlockSpec((1, ds, di), lambda l: (l, 0, 0)),
            pl.BlockSpec((1, 1, di), lambda l: (l, 0, 0)),
            pl.BlockSpec((1, dm, di), lambda l: (l, 0, 0)),
            pl.BlockSpec((1, 1, dm), lambda l: (0, 0, 0)),
        ],
        out_specs=pl.BlockSpec((m, dm), lambda l: (0, 0)),
        out_shape=jax.ShapeDtypeStruct((m, dm), jnp.bfloat16),
        scratch_shapes=[
            pltpu.VMEM((m, dm), jnp.float32),
            pltpu.VMEM((mb, 8, di), jnp.float32),
            pltpu.VMEM((mb, 8, di), jnp.bfloat16),
            pltpu.VMEM((mb, 8, di), jnp.float32),
            pltpu.VMEM((mb, 8, di), jnp.bfloat16),
            pltpu.VMEM((ds, m), jnp.float32),
            pltpu.VMEM((ds, m), jnp.float32),
            pltpu.VMEM((mb, ds, 8), jnp.float32),
            pltpu.VMEM((mb, ds, 8), jnp.float32),
        ],
        compiler_params=pltpu.CompilerParams(
            dimension_semantics=("arbitrary",),
            vmem_limit_bytes=_VMEM_LIM,
        ),
        name="mamba_layers",
        interpret=_INTERPRET,
    )(x0, norm_w, win_bf, cw, cb, wdtr_bf, wb_bf, wc_bf, wdt_bf, dtb,
      alogT, dmat, wo_bf, norm_f_w.reshape(1, 1, dm))


def _lmhead_body(h_ref, e_ref, o_ref):
    o_ref[...] = jax.lax.dot_general(
        h_ref[...], _bf(e_ref[...]), _CONTRACT_LAST,
        preferred_element_type=jnp.float32)


def _lmhead(hf, embed, *, vtile):
    m, dm = hf.shape
    v = embed.shape[0]
    nv = v // vtile
    return pl.pallas_call(
        _lmhead_body,
        grid=(nv,),
        in_specs=[
            pl.BlockSpec((m, dm), lambda i: (0, 0)),
            pl.BlockSpec((vtile, dm), lambda i: (i, 0)),
        ],
        out_specs=pl.BlockSpec((m, vtile), lambda i: (0, i)),
        out_shape=jax.ShapeDtypeStruct((m, v), jnp.float32),
        compiler_params=pltpu.CompilerParams(
            dimension_semantics=("arbitrary",),
            vmem_limit_bytes=_VMEM_LIM,
        ),
        name="lm_head",
        interpret=_INTERPRET,
    )(hf, embed)


def kernel(input_ids, embed, norm_w, in_proj_w, conv_w, conv_b, x_proj_w,
           dt_proj_w, dt_proj_b, A_log, D, out_proj_w, norm_f_w):
    bsz, seg = input_ids.shape
    v, dm = embed.shape
    nl, di, ds = A_log.shape
    dtr = dt_proj_w.shape[2]
    m = bsz * seg

    cw = jnp.swapaxes(conv_w[:, :, 0, :], 1, 2)
    alogT = jnp.swapaxes(A_log, 1, 2)
    wdtr = _bf(x_proj_w[:, :dtr, :])
    wb = _bf(x_proj_w[:, dtr:dtr + ds, :])
    wc = _bf(x_proj_w[:, dtr + ds:, :])
    win_bf = _bf(in_proj_w)
    wo_bf = _bf(out_proj_w)
    wdt_bf = _bf(dt_proj_w)

    x0 = _embed_gather(input_ids.reshape(m), embed)
    hf = _layers(x0, norm_w.reshape(nl, 1, dm), win_bf, cw,
                 conv_b.reshape(nl, 1, di), wdtr, wb, wc, wdt_bf,
                 dt_proj_b.reshape(nl, 1, di), alogT, D.reshape(nl, 1, di),
                 wo_bf, norm_f_w, seg=seg)
    logits = _lmhead(hf, embed, vtile=1280)
    return logits.reshape(bsz, seg, v)
